# baseline (speedup 1.0000x reference)
import functools

import jax
import jax.numpy as jnp
from jax import lax
from jax.experimental import pallas as pl
from jax.experimental.pallas import tpu as pltpu
from jax.experimental.pallas import tpu_sc as plsc

_NUM_LAYERS = 12
_POOL = 100
_NUM_PROMPTS = 10
_TOP_K = 4
_D = 768
_B = 64

_ROW = _NUM_PROMPTS * _D
_NROWS = _NUM_LAYERS * _B * _TOP_K
_VOCAB = _NUM_LAYERS * _POOL

_NC, _NS = 2, 16
_NW = _NC * _NS
_BPW = _NROWS // _NW
_CH = 8
_NCH = _BPW // _CH


def _topk_body(x_ref, ek_ref, idx_ref):
    neg = jnp.float32(-3.0e38)
    for l in range(_NUM_LAYERS):
        k_mat = ek_ref[l]
        kn = jnp.sqrt(jnp.sum(k_mat * k_mat, axis=1, keepdims=True))
        k_mat = k_mat / jnp.maximum(kn, 1e-12)
        q = x_ref[:, l, :]
        qn = jnp.sqrt(jnp.sum(q * q, axis=1, keepdims=True))
        q = q / jnp.maximum(qn, 1e-12)
        s = lax.dot_general(q, k_mat, (((1,), (1,)), ((), ())),
                            preferred_element_type=jnp.float32)
        col = lax.broadcasted_iota(jnp.int32, (_B, _POOL), 1)
        out_col = lax.broadcasted_iota(jnp.int32, (_B, 128), 1)
        acc = jnp.zeros((_B, 128), jnp.int32)
        for j in range(_TOP_K):
            m = jnp.max(s, axis=1, keepdims=True)
            idx = jnp.min(jnp.where(s >= m, col, _POOL), axis=1,
                          keepdims=True)
            acc = jnp.where(out_col == j, l * _POOL + idx, acc)
            s = jnp.where(col == idx, neg, s)
        idx_ref[l] = acc


def _compute_indices(x_query, e_k):
    out = pl.pallas_call(
        _topk_body,
        out_shape=jax.ShapeDtypeStruct((_NUM_LAYERS, _B, 128), jnp.int32),
    )(x_query, e_k)
    return out[:, :, :_TOP_K].reshape(-1)


def _gather_body(table_hbm, idx_hbm, out_hbm, idx_v, buf, sem):
    wid = lax.axis_index("s") * _NC + lax.axis_index("c")
    base = pl.multiple_of(wid * _BPW, _CH)
    pltpu.sync_copy(idx_hbm.at[pl.ds(base, _BPW)], idx_v)

    def chunk(c, carry):
        off = pl.multiple_of(base + c * _CH, _CH)
        pltpu.async_copy(table_hbm.at[idx_v.at[pl.ds(c * _CH, _CH)]],
                         buf, sem).wait()
        pltpu.sync_copy(buf, out_hbm.at[pl.ds(off, _CH)])
        return carry

    lax.fori_loop(0, _NCH, chunk, 0)


def _gather(table, flat_idx):
    mesh = plsc.VectorSubcoreMesh(core_axis_name="c", subcore_axis_name="s",
                                  num_cores=_NC, num_subcores=_NS)
    fn = pl.kernel(
        _gather_body,
        out_type=jax.ShapeDtypeStruct((_NROWS, _ROW), jnp.float32),
        mesh=mesh,
        scratch_types=[
            pltpu.VMEM((_BPW,), jnp.int32),
            pltpu.VMEM((_CH, _ROW), jnp.float32),
            pltpu.SemaphoreType.DMA,
        ],
    )
    return fn(table, flat_idx)


def kernel(x_query, e_p, e_k, vis_mark):
    flat_idx = _compute_indices(x_query, e_k)
    table = e_p.reshape(_VOCAB, _ROW)
    rows = _gather(table, flat_idx)
    p_return = rows.reshape(_NUM_LAYERS, _B, _TOP_K * _NUM_PROMPTS, _D)
    return (p_return, 0.0)
